# baseline (speedup 1.0000x reference)
import functools

import jax
import jax.numpy as jnp
from jax import lax
from jax.experimental import pallas as pl
from jax.experimental.pallas import tpu as pltpu
from jax.experimental.pallas import tpu_sc as plsc

B = 4096
L = 200
D = 128
LANES = 16
NCHUNK = D // LANES
CW = 128
NBUF = 5

_info = plsc.get_sparse_core_info()
_NC = _info.num_cores
_NS = _info.num_subcores
NW = _NC * _NS
RPW = B // NW
IPW = RPW * L
NCHUNKS = IPW // CW
PHASES = 25
NSUP = NCHUNKS // PHASES
ROWS_PER_SUP = PHASES * CW // L

_PHASE_END = []
for _p in range(PHASES):
    _ends = [g - CW * _p for g in range(CW * _p, CW * _p + CW)
             if g % L == L - 1]
    assert len(_ends) <= 1
    _PHASE_END.append(_ends[0] if _ends else None)


@functools.partial(
    pl.kernel,
    out_type=jax.ShapeDtypeStruct((B, D), jnp.float32),
    mesh=plsc.VectorSubcoreMesh(core_axis_name="c", subcore_axis_name="s"),
    scratch_types=[
        pltpu.VMEM((NCHUNKS, CW), jnp.int32),
        pltpu.VMEM((NBUF, CW, D), jnp.float32),
        pltpu.VMEM((2, D), jnp.float32),
        pltpu.SemaphoreType.DMA,
        pltpu.SemaphoreType.DMA,
        pltpu.SemaphoreType.DMA,
        pltpu.SemaphoreType.DMA,
        pltpu.SemaphoreType.DMA,
        pltpu.SemaphoreType.DMA,
    ],
)
def _bow_max_kernel(idx_hbm, table_hbm, out_hbm,
                    idx_v, rows_v, wbuf,
                    sem0, sem1, sem2, sem3, sem4, sem_out):
    wid = lax.axis_index("s") * _NC + lax.axis_index("c")
    base = wid * RPW

    pltpu.sync_copy(idx_hbm.at[wid], idx_v)

    sems = (sem0, sem1, sem2, sem3, sem4)

    def gather(c, slot):
        return pltpu.make_async_copy(
            table_hbm.at[idx_v.at[c]], rows_v.at[slot], sems[slot])

    def out_write(r, wslot):
        return pltpu.make_async_copy(
            wbuf.at[wslot], out_hbm.at[base + r], sem_out)

    def reduce_seg(slot, lo, hi, accs):
        if accs is None:
            accs = tuple(
                rows_v[slot, lo, pl.ds(ch * LANES, LANES)]
                for ch in range(NCHUNK))
            lo += 1
        if hi <= lo:
            return accs

        def body(j, a):
            return tuple(
                jnp.maximum(a[ch], rows_v[slot, j, pl.ds(ch * LANES, LANES)])
                for ch in range(NCHUNK))

        return lax.fori_loop(lo, hi, body, accs)

    def emit(r, k, accs):
        wslot = k % 2

        @pl.when(r >= 2)
        def _():
            out_write(r - 2, wslot).wait()

        for ch in range(NCHUNK):
            wbuf[wslot, pl.ds(ch * LANES, LANES)] = accs[ch]
        out_write(r, wslot).start()

    for p in range(NBUF - 1):
        gather(p, p).start()

    def superstep(s, _):
        accs = None
        k = 0
        for p in range(PHASES):
            c = PHASES * s + p
            slot = p % NBUF

            @pl.when(c + NBUF - 1 < NCHUNKS)
            def _():
                gather(c + NBUF - 1, (p + NBUF - 1) % NBUF).start()

            gather(c, slot).wait()

            e = _PHASE_END[p]
            if e is None:
                accs = reduce_seg(slot, 0, CW, accs)
            else:
                accs = reduce_seg(slot, 0, e + 1, accs)
                emit(ROWS_PER_SUP * s + k, k, accs)
                k += 1
                accs = None
                if e + 1 < CW:
                    accs = reduce_seg(slot, e + 1, CW, accs)
        assert k == ROWS_PER_SUP and accs is None
        return 0

    lax.fori_loop(0, NSUP, superstep, 0)

    out_write(RPW - 2, 0).wait()
    out_write(RPW - 1, 1).wait()


def kernel(inputs, emb_weight):
    idx3 = inputs.reshape(NW, NCHUNKS, CW)
    return _bow_max_kernel(idx3, emb_weight)


# speedup vs baseline: 1.0248x; 1.0248x over previous
import functools

import jax
import jax.numpy as jnp
from jax import lax
from jax.experimental import pallas as pl
from jax.experimental.pallas import tpu as pltpu
from jax.experimental.pallas import tpu_sc as plsc

B = 4096
L = 200
D = 128
LA = 128
LB = L - LA
LANES = 16
NCHUNK = D // LANES

_info = plsc.get_sparse_core_info()
_NC = _info.num_cores
_NS = _info.num_subcores
NW = _NC * _NS
RPW = B // NW


@functools.partial(
    pl.kernel,
    out_type=jax.ShapeDtypeStruct((B, D), jnp.float32),
    mesh=plsc.VectorSubcoreMesh(core_axis_name="c", subcore_axis_name="s"),
    scratch_types=[
        pltpu.VMEM((RPW, LA), jnp.int32),
        pltpu.VMEM((RPW, LB), jnp.int32),
        pltpu.VMEM((3, L, D), jnp.float32),
        pltpu.VMEM((RPW, D), jnp.float32),
        pltpu.SemaphoreType.DMA,
        pltpu.SemaphoreType.DMA,
        pltpu.SemaphoreType.DMA,
    ],
)
def _bow_max_kernel(idx_a_hbm, idx_b_hbm, table_hbm, out_hbm,
                    idx_a_v, idx_b_v, rows_v, out_v, sem0, sem1, sem2):
    wid = lax.axis_index("s") * _NC + lax.axis_index("c")
    base = wid * RPW

    pltpu.sync_copy(idx_a_hbm.at[pl.ds(base, RPW), :], idx_a_v)
    pltpu.sync_copy(idx_b_hbm.at[pl.ds(base, RPW), :], idx_b_v)

    sems = (sem0, sem1, sem2)
    NBUF = 3

    def gather(r, slot):
        sem = sems[slot]
        a = pltpu.make_async_copy(
            table_hbm.at[idx_a_v.at[r]], rows_v.at[slot, pl.ds(0, LA)], sem)
        b = pltpu.make_async_copy(
            table_hbm.at[idx_b_v.at[r]], rows_v.at[slot, pl.ds(LA, LB)], sem)
        return a, b

    def start_gather(r, slot):
        a, b = gather(r, slot)
        a.start()
        b.start()

    def wait_gather(r, slot):
        a, b = gather(r, slot)
        a.wait()
        b.wait()

    for p in range(NBUF - 1):
        start_gather(p, p)

    def do_row(r, slot):
        @pl.when(r < RPW - (NBUF - 1))
        def _():
            start_gather(r + NBUF - 1, (slot + NBUF - 1) % NBUF)

        wait_gather(r, slot)

        def body(j, accs):
            return tuple(
                jnp.maximum(accs[c], rows_v[slot, j, pl.ds(c * LANES, LANES)])
                for c in range(NCHUNK))

        init = tuple(
            rows_v[slot, 0, pl.ds(c * LANES, LANES)] for c in range(NCHUNK))
        accs = lax.fori_loop(1, L, body, init)
        for c in range(NCHUNK):
            out_v[r, pl.ds(c * LANES, LANES)] = accs[c]

    def outer(g, _):
        for b in range(NBUF):
            do_row(NBUF * g + b, b)
        return 0

    assert RPW % NBUF == 0 or True
    n_full = RPW // NBUF
    lax.fori_loop(0, n_full, outer, 0)
    for b in range(RPW - n_full * NBUF):
        do_row(n_full * NBUF + b, b)

    pltpu.sync_copy(out_v, out_hbm.at[pl.ds(base, RPW), :])


def kernel(inputs, emb_weight):
    idx_a = inputs[:, :LA]
    idx_b = inputs[:, LA:]
    return _bow_max_kernel(idx_a, idx_b, emb_weight)


# speedup vs baseline: 1.0277x; 1.0028x over previous
import functools

import jax
import jax.numpy as jnp
from jax import lax
from jax.experimental import pallas as pl
from jax.experimental.pallas import tpu as pltpu
from jax.experimental.pallas import tpu_sc as plsc

B = 4096
L = 200
D = 128
LA = 128
LB = L - LA
LANES = 16
NCHUNK = D // LANES
NBUF = 3

_info = plsc.get_sparse_core_info()
_NC = _info.num_cores
_NS = _info.num_subcores
NW = _NC * _NS
RPW = B // NW


@functools.partial(
    pl.kernel,
    out_type=jax.ShapeDtypeStruct((B, D), jnp.float32),
    mesh=plsc.VectorSubcoreMesh(core_axis_name="c", subcore_axis_name="s"),
    scratch_types=[
        pltpu.VMEM((RPW, L), jnp.int32),
        pltpu.VMEM((NBUF, L, D), jnp.float32),
        pltpu.VMEM((RPW, D), jnp.float32),
        pltpu.SemaphoreType.DMA,
        pltpu.SemaphoreType.DMA,
        pltpu.SemaphoreType.DMA,
    ],
)
def _bow_max_kernel(idx_hbm, table_hbm, out_hbm,
                    idx_v, rows_v, out_v, sem0, sem1, sem2):
    wid = lax.axis_index("s") * _NC + lax.axis_index("c")
    base = wid * RPW

    pltpu.sync_copy(idx_hbm.at[pl.ds(base, RPW), :], idx_v)

    sems = (sem0, sem1, sem2)

    def gather(r, slot):
        sem = sems[slot]
        a = pltpu.make_async_copy(
            table_hbm.at[idx_v.at[r, pl.ds(0, LA)]],
            rows_v.at[slot, pl.ds(0, LA)], sem)
        b = pltpu.make_async_copy(
            table_hbm.at[idx_v.at[r, pl.ds(LA, LB)]],
            rows_v.at[slot, pl.ds(LA, LB)], sem)
        return a, b

    def start_gather(r, slot):
        a, b = gather(r, slot)
        a.start()
        b.start()

    def wait_gather(r, slot):
        a, b = gather(r, slot)
        a.wait()
        b.wait()

    for p in range(NBUF - 1):
        start_gather(p, p)

    def do_row(r, slot):
        @pl.when(r < RPW - (NBUF - 1))
        def _():
            start_gather(r + NBUF - 1, (slot + NBUF - 1) % NBUF)

        wait_gather(r, slot)

        def body(j, accs):
            return tuple(
                jnp.maximum(accs[c], rows_v[slot, j, pl.ds(c * LANES, LANES)])
                for c in range(NCHUNK))

        init = tuple(
            rows_v[slot, 0, pl.ds(c * LANES, LANES)] for c in range(NCHUNK))
        accs = lax.fori_loop(1, L, body, init)
        for c in range(NCHUNK):
            out_v[r, pl.ds(c * LANES, LANES)] = accs[c]

    def outer(g, _):
        for b in range(NBUF):
            do_row(NBUF * g + b, b)
        return 0

    n_full = RPW // NBUF
    lax.fori_loop(0, n_full, outer, 0)
    for b in range(RPW - n_full * NBUF):
        do_row(n_full * NBUF + b, b)

    pltpu.sync_copy(out_v, out_hbm.at[pl.ds(base, RPW), :])


def kernel(inputs, emb_weight):
    return _bow_max_kernel(inputs, emb_weight)


# speedup vs baseline: 1.0340x; 1.0061x over previous
import functools

import jax
import jax.numpy as jnp
from jax import lax
from jax.experimental import pallas as pl
from jax.experimental.pallas import tpu as pltpu
from jax.experimental.pallas import tpu_sc as plsc

B = 4096
L = 200
D = 128
LA = 128
LB = L - LA
LANES = 16
NCHUNK = D // LANES
NBUF = 3

_info = plsc.get_sparse_core_info()
_NC = _info.num_cores
_NS = _info.num_subcores
NW = _NC * _NS
RPW = B // NW


@functools.partial(
    pl.kernel,
    out_type=jax.ShapeDtypeStruct((B, D), jnp.float32),
    mesh=plsc.VectorSubcoreMesh(core_axis_name="c", subcore_axis_name="s"),
    scratch_types=[
        pltpu.VMEM((RPW, L), jnp.int32),
        pltpu.VMEM((NBUF, L, D), jnp.float32),
        pltpu.VMEM((RPW, D), jnp.float32),
        pltpu.SemaphoreType.DMA,
        pltpu.SemaphoreType.DMA,
        pltpu.SemaphoreType.DMA,
        pltpu.SemaphoreType.DMA,
        pltpu.SemaphoreType.DMA,
        pltpu.SemaphoreType.DMA,
    ],
)
def _bow_max_kernel(idx_hbm, table_hbm, out_hbm,
                    idx_v, rows_v, out_v,
                    sem0a, sem1a, sem2a, sem0b, sem1b, sem2b):
    wid = lax.axis_index("s") * _NC + lax.axis_index("c")
    base = wid * RPW

    pltpu.sync_copy(idx_hbm.at[pl.ds(base, RPW), :], idx_v)

    sems_a = (sem0a, sem1a, sem2a)
    sems_b = (sem0b, sem1b, sem2b)

    def gather(r, slot):
        a = pltpu.make_async_copy(
            table_hbm.at[idx_v.at[r, pl.ds(0, LA)]],
            rows_v.at[slot, pl.ds(0, LA)], sems_a[slot])
        b = pltpu.make_async_copy(
            table_hbm.at[idx_v.at[r, pl.ds(LA, LB)]],
            rows_v.at[slot, pl.ds(LA, LB)], sems_b[slot])
        return a, b

    def start_gather(r, slot):
        a, b = gather(r, slot)
        a.start()
        b.start()

    for p in range(NBUF - 1):
        start_gather(p, p)

    def do_row(r, slot):
        @pl.when(r < RPW - (NBUF - 1))
        def _():
            start_gather(r + NBUF - 1, (slot + NBUF - 1) % NBUF)

        wa, wb = gather(r, slot)

        def body(j, accs):
            return tuple(
                jnp.maximum(accs[c], rows_v[slot, j, pl.ds(c * LANES, LANES)])
                for c in range(NCHUNK))

        wa.wait()
        init = tuple(
            rows_v[slot, 0, pl.ds(c * LANES, LANES)] for c in range(NCHUNK))
        accs = lax.fori_loop(1, LA, body, init)
        wb.wait()
        accs = lax.fori_loop(LA, L, body, accs)
        for c in range(NCHUNK):
            out_v[r, pl.ds(c * LANES, LANES)] = accs[c]

    def outer(g, _):
        for b in range(NBUF):
            do_row(NBUF * g + b, b)
        return 0

    n_full = RPW // NBUF
    lax.fori_loop(0, n_full, outer, 0)
    for b in range(RPW - n_full * NBUF):
        do_row(n_full * NBUF + b, b)

    pltpu.sync_copy(out_v, out_hbm.at[pl.ds(base, RPW), :])


def kernel(inputs, emb_weight):
    return _bow_max_kernel(inputs, emb_weight)
